# baseline (speedup 1.0000x reference)
import functools

import numpy as np

import jax
import jax.numpy as jnp
from jax import lax
from jax.experimental import pallas as pl
from jax.experimental.pallas import tpu as pltpu
from jax.experimental.pallas import tpu_sc as plsc

S = 128
E = 8
B = 1024
NC = 2
NS = 16
NW = NC * NS
TOK_W = B // NW
L = 16
NCH = S // L

OFF_W = 4 * S
OFF_WI = OFF_W + 2 * E * S
P_TOT = OFF_WI + 2 * E * S


_ANG = 2.0 * np.pi / S * ((np.arange(S)[:, None] * np.arange(S)[None, :]) % S)
_CW_NP = np.cos(_ANG).astype(np.float32)
_SW_NP = np.sin(_ANG).astype(np.float32)


def _prologue_body(mr_ref, mi_ref, tr_ref, ti_ref, cw_ref, sw_ref, p_ref):
    dot = functools.partial(
        lax.dot_general, preferred_element_type=jnp.float32,
        precision=lax.Precision.HIGHEST)
    dn_row = (((1,), (0,)), ((), ()))

    mr = mr_ref[:, :]
    mi = mi_ref[:, :]
    tr = tr_ref[:, :]
    ti = ti_ref[:, :]

    ex = jnp.exp(mr)
    cc = jnp.cos(mi)
    sn = jnp.sin(mi)
    zr = jnp.sum(ex * cc)
    zi = jnp.sum(ex * sn)

    ar = ex[0:1, :] * cc[0:1, :]
    ai = ex[0:1, :] * sn[0:1, :]
    br = ex[:, 0:1] * cc[:, 0:1]
    bi = ex[:, 0:1] * sn[:, 0:1]

    cw = cw_ref[:, :]
    sw = sw_ref[:, :]

    dn_col = (((0,), (0,)), ((), ()))
    vzr = dot(ar, cw, dimension_numbers=dn_row) + dot(
        ai, sw, dimension_numbers=dn_row)
    vzi = dot(ai, cw, dimension_numbers=dn_row) - dot(
        ar, sw, dimension_numbers=dn_row)
    uzr = dot(br, cw, dimension_numbers=dn_col) + dot(
        bi, sw, dimension_numbers=dn_col)
    uzi = dot(bi, cw, dimension_numbers=dn_col) - dot(
        br, sw, dimension_numbers=dn_col)

    den = zr * zr + zi * zi
    vr = (vzr * zr + vzi * zi) / den
    vi = (vzi * zr - vzr * zi) / den
    ur = (uzr * zr + uzi * zi) / den
    ui = (uzi * zr - uzr * zi) / den

    p_ref[0:4, :] = jnp.concatenate([vr, vi, ur, ui], axis=0)
    p_ref[4:4 + 2 * E, :] = 0.5 * jax.nn.sigmoid(tr)
    p_ref[4 + 2 * E:4 + 4 * E, :] = 0.5 * jax.nn.sigmoid(ti)


def _prologue(m_r, m_i, t_r, t_i):
    return pl.pallas_call(
        _prologue_body,
        out_shape=jax.ShapeDtypeStruct((4 + 4 * E, S), jnp.float32),
    )(m_r, m_i, t_r, t_i, jnp.asarray(_CW_NP), jnp.asarray(_SW_NP))


def _tree16(g):
    while len(g) > 1:
        g = [g[i] + g[i + 1] for i in range(0, len(g), 2)]
    return g[0]


def _sc_body(a_hbm, b_hbm, p_hbm, out_hbm, a_v, b_v, p_v, o_v, red_v, sum_v,
             e_v, sem):
    wid = lax.axis_index("c") * NS + lax.axis_index("s")
    base = wid * (TOK_W * S)
    cp_a = pltpu.async_copy(a_hbm.at[pl.ds(base, TOK_W * S)], a_v, sem)
    cp_b = pltpu.async_copy(b_hbm.at[pl.ds(base, TOK_W * S)], b_v, sem)
    cp_p = pltpu.async_copy(p_hbm, p_v, sem)
    cp_a.wait()
    cp_b.wait()
    cp_p.wait()

    lanes = lax.iota(jnp.int32, L)
    base16 = lanes * L
    cbase = OFF_W + (lanes & 7) * (2 * S)
    car = plsc.load_gather(p_v, [cbase])
    cbr = plsc.load_gather(p_v, [cbase + S])
    cai = plsc.load_gather(p_v, [cbase + 2 * E * S])
    cbi = plsc.load_gather(p_v, [cbase + 2 * E * S + S])

    def splat(ref, j):
        return plsc.load_gather(ref, [jnp.full((L,), j, jnp.int32)])

    TPB = 2

    def tok(t, carry):
        offs = [(TPB * t + i) * S for i in range(TPB)]
        acc = [jnp.zeros((L,), jnp.float32) for _ in range(6 * TPB)]
        for c in range(NCH):
            av = [a_v[pl.ds(o + c * L, L)] for o in offs]
            bv = [b_v[pl.ds(o + c * L, L)] for o in offs]
            vrc = p_v[pl.ds(0 * S + c * L, L)]
            vic = p_v[pl.ds(1 * S + c * L, L)]
            urc = p_v[pl.ds(2 * S + c * L, L)]
            uic = p_v[pl.ds(3 * S + c * L, L)]
            for i in range(TPB):
                j = 6 * i
                acc[j + 0] = acc[j + 0] + av[i]
                acc[j + 1] = acc[j + 1] + bv[i]
                acc[j + 2] = acc[j + 2] + bv[i] * vrc
                acc[j + 3] = acc[j + 3] + bv[i] * vic
                acc[j + 4] = acc[j + 4] + av[i] * urc
                acc[j + 5] = acc[j + 5] + av[i] * uic
        for h in range(TPB // 2):
            for j in range(6):
                red_v[pl.ds(h * 256 + (1 + j) * L, L)] = acc[12 * h + j]
                red_v[pl.ds(h * 256 + (9 + j) * L, L)] = acc[12 * h + 6 + j]
        for h in range(TPB // 2):
            sums = _tree16([
                plsc.load_gather(red_v, [base16 + h * 256 + k])
                for k in range(L)])
            sum_v[pl.ds(h * L, L)] = sums

        def route(sbase):
            sa = splat(sum_v, sbase + 1)
            sb = splat(sum_v, sbase + 2)
            par = splat(sum_v, sbase + 3)
            pai = splat(sum_v, sbase + 4)
            pbr = splat(sum_v, sbase + 5)
            pbi = splat(sum_v, sbase + 6)
            zar = sa * par
            zai = sa * pai
            zbr = sb * pbr
            zbi = sb * pbi
            re = zar * car + zai * cai + zbr * cbr + zbi * cbi
            im = zai * car - zar * cai + zbi * cbr - zbr * cbi
            sc = re * re + im * im
            sc = jnp.where(lanes < E, sc, -1.0)
            _, order = plsc.sort_key_val(sc, lanes, descending=True)
            return order

        orders = [route(8 * i) for i in range(TPB)]
        for i in range(TPB):
            e_v[pl.ds((2 * i) * L, L)] = orders[i]
            e_v[pl.ds((2 * i + 1) * L, L)] = orders[i]
        r1 = [OFF_W + splat(e_v, (2 * i + 1) * L) * (2 * S)
              for i in range(TPB)]
        r2 = [OFF_W + splat(e_v, 2 * i * L + 1) * (2 * S)
              for i in range(TPB)]
        for c in range(NCH):
            col = c * L + lanes
            for i in range(TPB):
                wa = plsc.load_gather(p_v, [r1[i] + col]) + plsc.load_gather(
                    p_v, [r2[i] + col])
                wb = plsc.load_gather(
                    p_v, [r1[i] + S + col]) + plsc.load_gather(
                    p_v, [r2[i] + S + col])
                ac = a_v[pl.ds(offs[i] + c * L, L)]
                bc = b_v[pl.ds(offs[i] + c * L, L)]
                o_v[pl.ds(offs[i] + c * L, L)] = wa * ac + wb * bc
        return carry

    lax.fori_loop(0, TOK_W // TPB, tok, jnp.int32(0))
    pltpu.sync_copy(o_v, out_hbm.at[pl.ds(base, TOK_W * S)])


@functools.cache
def _sc_call():
    return pl.kernel(
        _sc_body,
        compiler_params=pltpu.CompilerParams(needs_layout_passes=False),
        out_type=jax.ShapeDtypeStruct((B * S,), jnp.float32),
        mesh=plsc.VectorSubcoreMesh(
            core_axis_name="c", subcore_axis_name="s", num_cores=NC,
            num_subcores=NS),
        scratch_types=[
            pltpu.VMEM((TOK_W * S,), jnp.float32),
            pltpu.VMEM((TOK_W * S,), jnp.float32),
            pltpu.VMEM((P_TOT,), jnp.float32),
            pltpu.VMEM((TOK_W * S,), jnp.float32),
            pltpu.VMEM((2 * L * L,), jnp.float32),
            pltpu.VMEM((2 * L,), jnp.float32),
            pltpu.VMEM((8 * L,), jnp.int32),
            pltpu.SemaphoreType.DMA,
        ],
    )


def kernel(a, b, mask_ri, tokens_ri):
    mt = jnp.transpose(mask_ri, (2, 0, 1))
    tt = jnp.transpose(tokens_ri, (3, 0, 1, 2)).reshape(2, 2 * E, S)
    params = _prologue(mt[0], mt[1], tt[0], tt[1]).reshape(-1)
    out = _sc_call()(a.reshape(B * S), b.reshape(B * S), params)
    return out.reshape(B, 1, S)


# speedup vs baseline: 1.0450x; 1.0450x over previous
import functools

import numpy as np

import jax
import jax.numpy as jnp
from jax import lax
from jax.experimental import pallas as pl
from jax.experimental.pallas import tpu as pltpu
from jax.experimental.pallas import tpu_sc as plsc

S = 128
E = 8
B = 1024
NC = 2
NS = 16
NW = NC * NS
TOK_W = B // NW
L = 16
NCH = S // L

OFF_W = 4 * S
OFF_WI = OFF_W + 2 * E * S
P_TOT = OFF_WI + 2 * E * S


_ANG = 2.0 * np.pi / S * ((np.arange(S)[:, None] * np.arange(S)[None, :]) % S)
_CW_NP = np.cos(_ANG).astype(np.float32)
_SW_NP = np.sin(_ANG).astype(np.float32)


def _prologue_body(mr_ref, mi_ref, tr_ref, ti_ref, cw_ref, sw_ref, p_ref):
    dot = functools.partial(
        lax.dot_general, preferred_element_type=jnp.float32,
        precision=lax.Precision.HIGHEST)
    dn_row = (((1,), (0,)), ((), ()))

    mr = mr_ref[:, :]
    mi = mi_ref[:, :]
    tr = tr_ref[:, :]
    ti = ti_ref[:, :]

    ex = jnp.exp(mr)
    cc = jnp.cos(mi)
    sn = jnp.sin(mi)
    zr = jnp.sum(ex * cc)
    zi = jnp.sum(ex * sn)

    ar = ex[0:1, :] * cc[0:1, :]
    ai = ex[0:1, :] * sn[0:1, :]
    br = ex[:, 0:1] * cc[:, 0:1]
    bi = ex[:, 0:1] * sn[:, 0:1]

    cw = cw_ref[:, :]
    sw = sw_ref[:, :]

    dn_col = (((0,), (0,)), ((), ()))
    vzr = dot(ar, cw, dimension_numbers=dn_row) + dot(
        ai, sw, dimension_numbers=dn_row)
    vzi = dot(ai, cw, dimension_numbers=dn_row) - dot(
        ar, sw, dimension_numbers=dn_row)
    uzr = dot(br, cw, dimension_numbers=dn_col) + dot(
        bi, sw, dimension_numbers=dn_col)
    uzi = dot(bi, cw, dimension_numbers=dn_col) - dot(
        br, sw, dimension_numbers=dn_col)

    den = zr * zr + zi * zi
    vr = (vzr * zr + vzi * zi) / den
    vi = (vzi * zr - vzr * zi) / den
    ur = (uzr * zr + uzi * zi) / den
    ui = (uzi * zr - uzr * zi) / den

    p_ref[0:4, :] = jnp.concatenate([vr, vi, ur, ui], axis=0)
    p_ref[4:4 + 2 * E, :] = 0.5 * jax.nn.sigmoid(tr)
    p_ref[4 + 2 * E:4 + 4 * E, :] = 0.5 * jax.nn.sigmoid(ti)


def _prologue(m_r, m_i, t_r, t_i):
    return pl.pallas_call(
        _prologue_body,
        out_shape=jax.ShapeDtypeStruct((4 + 4 * E, S), jnp.float32),
    )(m_r, m_i, t_r, t_i, jnp.asarray(_CW_NP), jnp.asarray(_SW_NP))


def _tree16(g):
    while len(g) > 1:
        g = [g[i] + g[i + 1] for i in range(0, len(g), 2)]
    return g[0]


def _sc_body(a_hbm, b_hbm, p_hbm, out_hbm, a_v, b_v, p_v, o_v, red_v, sum_v,
             e_v, sem):
    wid = lax.axis_index("c") * NS + lax.axis_index("s")
    base = wid * (TOK_W * S)
    cp_a = pltpu.async_copy(a_hbm.at[pl.ds(base, TOK_W * S)], a_v, sem)
    cp_b = pltpu.async_copy(b_hbm.at[pl.ds(base, TOK_W * S)], b_v, sem)
    cp_p = pltpu.async_copy(p_hbm, p_v, sem)
    cp_a.wait()
    cp_b.wait()
    cp_p.wait()

    lanes = lax.iota(jnp.int32, L)
    base16 = lanes * L
    cbase = OFF_W + (lanes & 7) * (2 * S)
    car = plsc.load_gather(p_v, [cbase])
    cbr = plsc.load_gather(p_v, [cbase + S])
    cai = plsc.load_gather(p_v, [cbase + 2 * E * S])
    cbi = plsc.load_gather(p_v, [cbase + 2 * E * S + S])

    def splat(ref, j):
        return plsc.load_gather(ref, [jnp.full((L,), j, jnp.int32)])

    TPB = 2

    def tok(t, carry):
        offs = [(TPB * t + i) * S for i in range(TPB)]
        acc = [jnp.zeros((L,), jnp.float32) for _ in range(6 * TPB)]
        for c in range(NCH):
            av = [a_v[pl.ds(o + c * L, L)] for o in offs]
            bv = [b_v[pl.ds(o + c * L, L)] for o in offs]
            vrc = p_v[pl.ds(0 * S + c * L, L)]
            vic = p_v[pl.ds(1 * S + c * L, L)]
            urc = p_v[pl.ds(2 * S + c * L, L)]
            uic = p_v[pl.ds(3 * S + c * L, L)]
            for i in range(TPB):
                j = 6 * i
                acc[j + 0] = acc[j + 0] + av[i]
                acc[j + 1] = acc[j + 1] + bv[i]
                acc[j + 2] = acc[j + 2] + bv[i] * vrc
                acc[j + 3] = acc[j + 3] + bv[i] * vic
                acc[j + 4] = acc[j + 4] + av[i] * urc
                acc[j + 5] = acc[j + 5] + av[i] * uic
        for h in range(TPB // 2):
            for j in range(6):
                red_v[pl.ds(h * 256 + (1 + j) * L, L)] = acc[12 * h + j]
                red_v[pl.ds(h * 256 + (9 + j) * L, L)] = acc[12 * h + 6 + j]
        for h in range(TPB // 2):
            sums = _tree16([
                plsc.load_gather(red_v, [base16 + h * 256 + k])
                for k in range(L)])
            sum_v[pl.ds(h * L, L)] = sums

        def route(sbase):
            sa = splat(sum_v, sbase + 1)
            sb = splat(sum_v, sbase + 2)
            par = splat(sum_v, sbase + 3)
            pai = splat(sum_v, sbase + 4)
            pbr = splat(sum_v, sbase + 5)
            pbi = splat(sum_v, sbase + 6)
            zar = sa * par
            zai = sa * pai
            zbr = sb * pbr
            zbi = sb * pbi
            re = zar * car + zai * cai + zbr * cbr + zbi * cbi
            im = zai * car - zar * cai + zbi * cbr - zbr * cbi
            sc = re * re + im * im
            sc = jnp.where(lanes < E, sc, -1.0)
            _, order = plsc.sort_key_val(sc, lanes, descending=True)
            return order

        orders = [route(8 * i) for i in range(TPB)]
        for i in range(TPB):
            e_v[pl.ds((2 * i) * L, L)] = orders[i]
            e_v[pl.ds((2 * i + 1) * L, L)] = orders[i]
        r1 = [OFF_W + splat(e_v, (2 * i + 1) * L) * (2 * S)
              for i in range(TPB)]
        r2 = [OFF_W + splat(e_v, 2 * i * L + 1) * (2 * S)
              for i in range(TPB)]
        for c in range(NCH):
            col = c * L + lanes
            was = [plsc.load_gather(p_v, [r1[i] + col]) + plsc.load_gather(
                p_v, [r2[i] + col]) for i in range(TPB)]
            wbs = [plsc.load_gather(
                p_v, [r1[i] + S + col]) + plsc.load_gather(
                p_v, [r2[i] + S + col]) for i in range(TPB)]
            for i in range(TPB):
                ac = a_v[pl.ds(offs[i] + c * L, L)]
                bc = b_v[pl.ds(offs[i] + c * L, L)]
                o_v[pl.ds(offs[i] + c * L, L)] = was[i] * ac + wbs[i] * bc
        return carry

    lax.fori_loop(0, TOK_W // TPB, tok, jnp.int32(0))
    pltpu.sync_copy(o_v, out_hbm.at[pl.ds(base, TOK_W * S)])


@functools.cache
def _sc_call():
    return pl.kernel(
        _sc_body,
        compiler_params=pltpu.CompilerParams(needs_layout_passes=False),
        out_type=jax.ShapeDtypeStruct((B * S,), jnp.float32),
        mesh=plsc.VectorSubcoreMesh(
            core_axis_name="c", subcore_axis_name="s", num_cores=NC,
            num_subcores=NS),
        scratch_types=[
            pltpu.VMEM((TOK_W * S,), jnp.float32),
            pltpu.VMEM((TOK_W * S,), jnp.float32),
            pltpu.VMEM((P_TOT,), jnp.float32),
            pltpu.VMEM((TOK_W * S,), jnp.float32),
            pltpu.VMEM((2 * L * L,), jnp.float32),
            pltpu.VMEM((2 * L,), jnp.float32),
            pltpu.VMEM((8 * L,), jnp.int32),
            pltpu.SemaphoreType.DMA,
        ],
    )


def kernel(a, b, mask_ri, tokens_ri):
    m_r = mask_ri[..., 0]
    m_i = mask_ri[..., 1]
    t_r = tokens_ri[..., 0].reshape(2 * E, S)
    t_i = tokens_ri[..., 1].reshape(2 * E, S)
    params = _prologue(m_r, m_i, t_r, t_i).reshape(-1)
    out = _sc_call()(a.reshape(B * S), b.reshape(B * S), params)
    return out.reshape(B, 1, S)
